# baseline (speedup 1.0000x reference)
import functools

import jax
import jax.numpy as jnp
from jax import lax
from jax.experimental import pallas as pl
from jax.experimental.pallas import tpu as pltpu
from jax.experimental.pallas import tpu_sc as plsc

VOCAB = 100000
D_MODEL = 64
BATCH = 1024
N_BLK = 2048

_TR_W = 512
_TR_BLOCKS = 99
_PAIR_OFF = 97 * _TR_W
_SPLIT = 50000


def _tr_body(a_ref, b_ref, out_ref):
    eye = jnp.eye(D_MODEL, dtype=jnp.float32)
    out_ref[:, 0:D_MODEL] = lax.dot_general(
        a_ref[...], eye, (((0,), (0,)), ((), ())),
        preferred_element_type=jnp.float32,
    )
    out_ref[:, D_MODEL:] = lax.dot_general(
        b_ref[...], eye, (((0,), (0,)), ((), ())),
        preferred_element_type=jnp.float32,
    )


def _tc_pair_table(tt):
    return pl.pallas_call(
        _tr_body,
        grid=(_TR_BLOCKS,),
        in_specs=[
            pl.BlockSpec((D_MODEL, _TR_W), lambda j: (0, j)),
            pl.BlockSpec((D_MODEL, _TR_W), lambda j: (0, j + _PAIR_OFF // _TR_W)),
        ],
        out_specs=pl.BlockSpec((_TR_W, 2 * D_MODEL), lambda j: (j, 0)),
        out_shape=jax.ShapeDtypeStruct((_TR_BLOCKS * _TR_W, 2 * D_MODEL), jnp.float32),
    )(tt, tt)


@functools.lru_cache(maxsize=None)
def _make_sc_gather_pairs():
    info = plsc.get_sparse_core_info()
    nw = info.num_cores * info.num_subcores
    b_per_w = BATCH // nw
    mesh = plsc.VectorSubcoreMesh(core_axis_name="c", subcore_axis_name="s")

    @functools.partial(
        pl.kernel,
        mesh=mesh,
        out_type=jax.ShapeDtypeStruct((BATCH, 2 * D_MODEL), jnp.float32),
        scratch_types=[
            pltpu.VMEM((BATCH,), jnp.int32),
            pltpu.VMEM((b_per_w, 2 * D_MODEL), jnp.float32),
            pltpu.SemaphoreType.DMA,
        ],
    )
    def gather(t2_hbm, idx_hbm, out_hbm, idx_v, rows_v, sem):
        wid = lax.axis_index("s") * info.num_cores + lax.axis_index("c")
        base = wid * b_per_w
        pltpu.sync_copy(idx_hbm, idx_v)
        pltpu.async_copy(
            t2_hbm.at[idx_v.at[pl.ds(base, b_per_w)]], rows_v, sem
        ).wait()
        pltpu.sync_copy(rows_v, out_hbm.at[pl.ds(base, b_per_w)])

    return gather


def _matmul_body(wt_ref, e128_ref, p_ref, out_ref, esel_ref):
    @pl.when(pl.program_id(0) == 0)
    def _():
        e = e128_ref[...]
        pvec = p_ref[...]
        lo = e[:, :D_MODEL]
        hi = e[:, D_MODEL:]
        esel_ref[...] = lo + (hi - lo) * pvec

    out_ref[...] = lax.dot_general(
        wt_ref[...],
        esel_ref[...],
        dimension_numbers=(((0,), (1,)), ((), ())),
        preferred_element_type=jnp.float32,
    )


def _tc_matmul_t(wt, e128, p):
    return pl.pallas_call(
        _matmul_body,
        grid=(pl.cdiv(VOCAB, N_BLK),),
        in_specs=[
            pl.BlockSpec((D_MODEL, N_BLK), lambda i: (0, i)),
            pl.BlockSpec((BATCH, 2 * D_MODEL), lambda i: (0, 0)),
            pl.BlockSpec((BATCH, 1), lambda i: (0, 0)),
        ],
        out_specs=pl.BlockSpec((N_BLK, BATCH), lambda i: (i, 0)),
        out_shape=jax.ShapeDtypeStruct((VOCAB, BATCH), jnp.float32),
        scratch_shapes=[pltpu.VMEM((BATCH, D_MODEL), jnp.float32)],
    )(wt, e128, p)


def kernel(x, emb_table, W):
    xi = x.astype(jnp.int32)
    t2 = _tc_pair_table(jnp.transpose(emb_table))
    k = jnp.where(xi < _SPLIT, xi, xi - _PAIR_OFF)
    p = (xi >= _SPLIT).astype(jnp.float32)[:, None]
    e128 = _make_sc_gather_pairs()(t2, k)
    out_t = _tc_matmul_t(jnp.transpose(W), e128, p)
    return jnp.transpose(out_t)


# speedup vs baseline: 1.2193x; 1.2193x over previous
import functools

import jax
import jax.numpy as jnp
from jax import lax
from jax.experimental import pallas as pl
from jax.experimental.pallas import tpu as pltpu
from jax.experimental.pallas import tpu_sc as plsc

VOCAB = 100000
D_MODEL = 64
BATCH = 1024
N_BLK = 4096

_TR_W = 2048
_TR_BLOCKS = 25
_PAIR_OFF = 24 * _TR_W
_SPLIT = 50000


def _tr_body(a_ref, b_ref, out_ref):
    eye = jnp.eye(D_MODEL, dtype=jnp.float32)
    out_ref[:, 0:D_MODEL] = lax.dot_general(
        a_ref[...], eye, (((0,), (0,)), ((), ())),
        preferred_element_type=jnp.float32,
    )
    out_ref[:, D_MODEL:] = lax.dot_general(
        b_ref[...], eye, (((0,), (0,)), ((), ())),
        preferred_element_type=jnp.float32,
    )


def _tc_pair_table(tt):
    return pl.pallas_call(
        _tr_body,
        grid=(_TR_BLOCKS,),
        in_specs=[
            pl.BlockSpec((D_MODEL, _TR_W), lambda j: (0, j)),
            pl.BlockSpec((D_MODEL, _TR_W), lambda j: (0, j + _PAIR_OFF // _TR_W)),
        ],
        out_specs=pl.BlockSpec((_TR_W, 2 * D_MODEL), lambda j: (j, 0)),
        out_shape=jax.ShapeDtypeStruct((_TR_BLOCKS * _TR_W, 2 * D_MODEL), jnp.float32),
    )(tt, tt)


@functools.lru_cache(maxsize=None)
def _make_sc_gather_pairs():
    info = plsc.get_sparse_core_info()
    nw = info.num_cores * info.num_subcores
    b_per_w = BATCH // nw
    mesh = plsc.VectorSubcoreMesh(core_axis_name="c", subcore_axis_name="s")

    @functools.partial(
        pl.kernel,
        mesh=mesh,
        out_type=jax.ShapeDtypeStruct((BATCH, 2 * D_MODEL), jnp.float32),
        scratch_types=[
            pltpu.VMEM((BATCH,), jnp.int32),
            pltpu.VMEM((b_per_w, 2 * D_MODEL), jnp.float32),
            pltpu.SemaphoreType.DMA,
        ],
    )
    def gather(t2_hbm, idx_hbm, out_hbm, idx_v, rows_v, sem):
        wid = lax.axis_index("s") * info.num_cores + lax.axis_index("c")
        base = wid * b_per_w
        pltpu.sync_copy(idx_hbm, idx_v)
        pltpu.async_copy(
            t2_hbm.at[idx_v.at[pl.ds(base, b_per_w)]], rows_v, sem
        ).wait()
        pltpu.sync_copy(rows_v, out_hbm.at[pl.ds(base, b_per_w)])

    return gather


def _matmul_body(wt_ref, e128_ref, p_ref, out_ref, esel_ref):
    @pl.when(pl.program_id(0) == 0)
    def _():
        e = e128_ref[...]
        pvec = p_ref[...]
        lo = e[:, :D_MODEL]
        hi = e[:, D_MODEL:]
        esel_ref[...] = lo + (hi - lo) * pvec

    out_ref[...] = lax.dot_general(
        wt_ref[...],
        esel_ref[...],
        dimension_numbers=(((0,), (1,)), ((), ())),
        preferred_element_type=jnp.float32,
    )


def _tc_matmul_t(wt, e128, p):
    return pl.pallas_call(
        _matmul_body,
        grid=(pl.cdiv(VOCAB, N_BLK),),
        in_specs=[
            pl.BlockSpec((D_MODEL, N_BLK), lambda i: (0, i)),
            pl.BlockSpec((BATCH, 2 * D_MODEL), lambda i: (0, 0)),
            pl.BlockSpec((BATCH, 1), lambda i: (0, 0)),
        ],
        out_specs=pl.BlockSpec((N_BLK, BATCH), lambda i: (i, 0)),
        out_shape=jax.ShapeDtypeStruct((VOCAB, BATCH), jnp.float32),
        scratch_shapes=[pltpu.VMEM((BATCH, D_MODEL), jnp.float32)],
    )(wt, e128, p)


def kernel(x, emb_table, W):
    xi = x.astype(jnp.int32)
    t2 = _tc_pair_table(jnp.transpose(emb_table))
    k = jnp.where(xi < _SPLIT, xi, xi - _PAIR_OFF)
    p = (xi >= _SPLIT).astype(jnp.float32)[:, None]
    e128 = _make_sc_gather_pairs()(t2, k)
    out_t = _tc_matmul_t(jnp.transpose(W), e128, p)
    return jnp.transpose(out_t)


# speedup vs baseline: 1.2586x; 1.0322x over previous
import functools

import jax
import jax.numpy as jnp
from jax import lax
from jax.experimental import pallas as pl
from jax.experimental.pallas import tpu as pltpu
from jax.experimental.pallas import tpu_sc as plsc

VOCAB = 100000
D_MODEL = 64
BATCH = 1024
N_BLK = 4096

_TR_W = 4096
_TR_BLOCKS = 13
_PAIR_OFF = 12 * _TR_W
_SPLIT = 50000


def _tr_body(a_ref, b_ref, out_ref):
    eye = jnp.eye(D_MODEL, dtype=jnp.float32)
    out_ref[:, 0:D_MODEL] = lax.dot_general(
        a_ref[...], eye, (((0,), (0,)), ((), ())),
        preferred_element_type=jnp.float32,
    )
    out_ref[:, D_MODEL:] = lax.dot_general(
        b_ref[...], eye, (((0,), (0,)), ((), ())),
        preferred_element_type=jnp.float32,
    )


def _tc_pair_table(tt):
    return pl.pallas_call(
        _tr_body,
        grid=(_TR_BLOCKS,),
        in_specs=[
            pl.BlockSpec((D_MODEL, _TR_W), lambda j: (0, j)),
            pl.BlockSpec((D_MODEL, _TR_W), lambda j: (0, j + _PAIR_OFF // _TR_W)),
        ],
        out_specs=pl.BlockSpec((_TR_W, 2 * D_MODEL), lambda j: (j, 0)),
        out_shape=jax.ShapeDtypeStruct((_TR_BLOCKS * _TR_W, 2 * D_MODEL), jnp.float32),
    )(tt, tt)


@functools.lru_cache(maxsize=None)
def _make_sc_gather_pairs():
    info = plsc.get_sparse_core_info()
    nw = info.num_cores * info.num_subcores
    b_per_w = BATCH // nw
    mesh = plsc.VectorSubcoreMesh(core_axis_name="c", subcore_axis_name="s")

    @functools.partial(
        pl.kernel,
        mesh=mesh,
        out_type=jax.ShapeDtypeStruct((BATCH, 2 * D_MODEL), jnp.float32),
        scratch_types=[
            pltpu.VMEM((BATCH,), jnp.int32),
            pltpu.VMEM((b_per_w, 2 * D_MODEL), jnp.float32),
            pltpu.SemaphoreType.DMA,
        ],
    )
    def gather(t2_hbm, idx_hbm, out_hbm, idx_v, rows_v, sem):
        wid = lax.axis_index("s") * info.num_cores + lax.axis_index("c")
        base = wid * b_per_w
        pltpu.sync_copy(idx_hbm, idx_v)
        pltpu.async_copy(
            t2_hbm.at[idx_v.at[pl.ds(base, b_per_w)]], rows_v, sem
        ).wait()
        pltpu.sync_copy(rows_v, out_hbm.at[pl.ds(base, b_per_w)])

    return gather


def _matmul_body(wt_ref, e128_ref, p_ref, out_ref, esel_ref):
    @pl.when(pl.program_id(0) == 0)
    def _():
        e = e128_ref[...]
        pvec = p_ref[...]
        lo = e[:, :D_MODEL]
        hi = e[:, D_MODEL:]
        esel_ref[...] = lo + (hi - lo) * pvec

    out_ref[...] = lax.dot_general(
        wt_ref[...],
        esel_ref[...],
        dimension_numbers=(((0,), (1,)), ((), ())),
        preferred_element_type=jnp.float32,
    )


def _tc_matmul_t(wt, e128, p):
    return pl.pallas_call(
        _matmul_body,
        grid=(pl.cdiv(VOCAB, N_BLK),),
        in_specs=[
            pl.BlockSpec((D_MODEL, N_BLK), lambda i: (0, i)),
            pl.BlockSpec((BATCH, 2 * D_MODEL), lambda i: (0, 0)),
            pl.BlockSpec((BATCH, 1), lambda i: (0, 0)),
        ],
        out_specs=pl.BlockSpec((N_BLK, BATCH), lambda i: (i, 0)),
        out_shape=jax.ShapeDtypeStruct((VOCAB, BATCH), jnp.float32),
        scratch_shapes=[pltpu.VMEM((BATCH, D_MODEL), jnp.float32)],
    )(wt, e128, p)


def kernel(x, emb_table, W):
    xi = x.astype(jnp.int32)
    t2 = _tc_pair_table(jnp.transpose(emb_table))
    k = jnp.where(xi < _SPLIT, xi, xi - _PAIR_OFF)
    p = (xi >= _SPLIT).astype(jnp.float32)[:, None]
    e128 = _make_sc_gather_pairs()(t2, k)
    out_t = _tc_matmul_t(jnp.transpose(W), e128, p)
    return jnp.transpose(out_t)
